# baseline (speedup 1.0000x reference)
import math

import jax
import jax.numpy as jnp
from jax import lax
from jax.experimental import pallas as pl
from jax.experimental.pallas import tpu as pltpu
from jax.experimental.pallas import tpu_sc as plsc

_VOCAB = 1000000
_D = 64
_B = 4096
_S = 200

_NC = 2
_NS = 16
_NW = _NC * _NS

_ROWS = _B * _S
_RPW = _ROWS // _NW
_CH = _S
_NCH = _RPW // _CH
_HALF = _CH // 2
_XROWS = _ROWS // _HALF


def _pe_table():
    pos = jnp.arange(0, _S, dtype=jnp.float32)[:, None]
    ang = pos * jnp.exp(
        -jnp.arange(0, _D, 2, dtype=jnp.float32) * math.log(1000.0) / _D)
    pe = jnp.zeros((_S, _D), dtype=jnp.float32)
    pe = pe.at[:, 0::2].set(jnp.sin(ang))
    pe = pe.at[:, 1::2].set(jnp.cos(ang))
    return pe.reshape(2 * _NCH // _NCH, _HALF, _D)


def _body(x2, W, pe, out, pe_v, idx_v, gbuf, obuf,
          sem_g0, sem_g1, sem_o0, sem_o1):
    sem_g = (sem_g0, sem_g1)
    sem_o = (sem_o0, sem_o1)
    cid = lax.axis_index("c")
    sid = lax.axis_index("s")
    wid = sid * _NC + cid
    xbase = wid * (_RPW // _HALF)

    pltpu.sync_copy(pe, pe_v)

    def fire_gathers(b, cc):
        r0 = xbase + cc * 2
        pltpu.sync_copy(x2.at[pl.ds(r0, 2)], idx_v.at[b])
        for j in range(2):
            pltpu.async_copy(W.at[idx_v.at[b, j]], gbuf.at[b, j], sem_g[b])

    def wait_gathers(b):
        for j in range(2):
            pltpu.make_async_copy(W.at[idx_v.at[b, j]], gbuf.at[b, j],
                                  sem_g[b]).wait()

    def fire_store(b, cc):
        r0 = xbase + cc * 2
        pltpu.async_copy(obuf.at[b], out.at[pl.ds(r0, 2)], sem_o[b])

    def wait_store(b):
        pltpu.make_async_copy(obuf.at[b], out.at[pl.ds(0, 2)], sem_o[b]).wait()

    def add_pe(b):
        @pl.loop(0, _HALF)
        def _(r):
            for j in range(2):
                for k in range(_D // 16):
                    sl = pl.ds(k * 16, 16)
                    obuf[b, j, r, sl] = gbuf[b, j, r, sl] + pe_v[j, r, sl]

    fire_gathers(0, 0)
    fire_gathers(1, 1)

    @pl.loop(0, _NCH, step=2)
    def _(c):
        for b in range(2):
            cc = c + b
            wait_gathers(b)

            @pl.when(cc >= 2)
            def _():
                wait_store(b)

            add_pe(b)

            @pl.when(cc + 2 < _NCH)
            def _():
                fire_gathers(b, cc + 2)

            fire_store(b, cc)

    wait_store(0)
    wait_store(1)


def kernel(x, W):
    pe = _pe_table()
    x2 = x.astype(jnp.int32).reshape(_XROWS, _HALF)
    call = pl.kernel(
        _body,
        out_type=jax.ShapeDtypeStruct((_XROWS, _HALF, _D), jnp.float32),
        mesh=plsc.VectorSubcoreMesh(core_axis_name="c", subcore_axis_name="s"),
        compiler_params=pltpu.CompilerParams(use_tc_tiling_on_sc=False),
        scratch_types=[
            pltpu.VMEM((2, _HALF, _D), jnp.float32),
            pltpu.VMEM((2, 2, _HALF), jnp.int32),
            pltpu.VMEM((2, 2, _HALF, _D), jnp.float32),
            pltpu.VMEM((2, 2, _HALF, _D), jnp.float32),
            pltpu.SemaphoreType.DMA,
            pltpu.SemaphoreType.DMA,
            pltpu.SemaphoreType.DMA,
            pltpu.SemaphoreType.DMA,
        ],
    )
    out = call(x2, W, pe)
    return out.reshape(_B, _S, _D)
